# Initial kernel scaffold; baseline (speedup 1.0000x reference)
import jax
import jax.numpy as jnp
from jax.experimental import pallas as pl


def kernel(x, edge_index, batch, edge_attr, W_ne, b_ne, W_ee, b_ee, W_conv, b_conv, bn_gamma, bn_beta, fc1_W, fc1_b, fc2_W, fc2_b, fc3_W, fc3_b):
    raise NotImplementedError("write your pallas kernel here")


# speedup vs baseline: 2.2098x; 2.2098x over previous
import functools
import numpy as np
import jax
import jax.numpy as jnp
from jax.experimental import pallas as pl
from jax.experimental.pallas import tpu as pltpu

N = 10000
E = 160000
H = 256
NG = 128
NLAYERS = 3
NBLK = 400
NNB = N // NBLK
EBLK = 1000
_DEG_HIST = np.array([0., 1000., 3000., 4000., 1500., 500.])
_b = np.arange(len(_DEG_HIST))
AVG_DEG_LOG = float((np.log(_b + 1.0) * _DEG_HIST).sum() / _DEG_HIST.sum())


def _mm_body(a_ref, w_ref, b_ref, y_ref):
    y_ref[...] = jnp.dot(a_ref[...], w_ref[...],
                         preferred_element_type=jnp.float32) + b_ref[...]


def _matmul_bias(a, w, b, blk):
    M, K = a.shape
    F = w.shape[1]
    return pl.pallas_call(
        _mm_body,
        grid=(M // blk,),
        in_specs=[pl.BlockSpec((blk, K), lambda i: (i, 0)),
                  pl.BlockSpec((K, F), lambda i: (0, 0)),
                  pl.BlockSpec((1, F), lambda i: (0, 0))],
        out_specs=pl.BlockSpec((blk, F), lambda i: (i, 0)),
        out_shape=jax.ShapeDtypeStruct((M, F), jnp.float32),
    )(a, w, b.reshape(1, F))


def _layer_body(h_ref, deg_ref, ssum_ref, smin_ref, smax_ref, sssq_ref,
                e4_ref, wdst_ref, wsrc_ref, wea_ref, c_ref,
                out_ref, psum_ref, psq_ref):
    deg = deg_ref[...]
    cntc = jnp.maximum(deg, 1.0)
    inv = 1.0 / cntc
    ld = jnp.log(jnp.maximum(deg, 1.0) + 1.0)
    amp = ld / AVG_DEG_LOG
    att = AVG_DEG_LOG / ld
    dmask = deg > 0

    mean = ssum_ref[...] * inv
    std = jnp.sqrt(jax.nn.relu(sssq_ref[...] * inv - mean * mean) + 1e-5)
    s4 = jnp.concatenate([
        mean,
        jnp.where(dmask, smin_ref[...], 0.0),
        jnp.where(dmask, smax_ref[...], 0.0),
        std,
    ], axis=1)
    hm = jnp.where(dmask, h_ref[...], 0.0)

    p = (jnp.dot(hm, wdst_ref[...], preferred_element_type=jnp.float32)
         + jnp.dot(s4, wsrc_ref[...], preferred_element_type=jnp.float32)
         + jnp.dot(e4_ref[...], wea_ref[...], preferred_element_type=jnp.float32))
    c = c_ref[...]
    out = ((p[:, :H] + c[0:1, :])
           + amp * (p[:, H:2 * H] + c[1:2, :])
           + att * (p[:, 2 * H:] + c[2:3, :]))
    out_ref[...] = out
    psum_ref[...] = jnp.sum(out, axis=0, keepdims=True)[None]
    psq_ref[...] = jnp.sum(out * out, axis=0, keepdims=True)[None]


def _layer_matmul(h, deg2, ssum, smin, smax, sssq, e4, wdst, wsrc, wea, c):
    nb = pl.BlockSpec((NBLK, H), lambda i: (i, 0))
    return pl.pallas_call(
        _layer_body,
        grid=(NNB,),
        in_specs=[nb,
                  pl.BlockSpec((NBLK, 1), lambda i: (i, 0)),
                  nb, nb, nb, nb,
                  pl.BlockSpec((NBLK, 4 * H), lambda i: (i, 0)),
                  pl.BlockSpec((H, 3 * H), lambda i: (0, 0)),
                  pl.BlockSpec((4 * H, 3 * H), lambda i: (0, 0)),
                  pl.BlockSpec((4 * H, 3 * H), lambda i: (0, 0)),
                  pl.BlockSpec((3, H), lambda i: (0, 0))],
        out_specs=[nb,
                   pl.BlockSpec((1, 1, H), lambda i: (i, 0, 0)),
                   pl.BlockSpec((1, 1, H), lambda i: (i, 0, 0))],
        out_shape=[jax.ShapeDtypeStruct((N, H), jnp.float32),
                   jax.ShapeDtypeStruct((NNB, 1, H), jnp.float32),
                   jax.ShapeDtypeStruct((NNB, 1, H), jnp.float32)],
    )(h, deg2, ssum, smin, smax, sssq, e4, wdst, wsrc, wea, c)


def _bn_body(out_ref, psum_ref, psq_ref, gam_ref, bet_ref, h_ref, hnew_ref):
    mu = jnp.sum(psum_ref[...], axis=0) * (1.0 / N)
    msq = jnp.sum(psq_ref[...], axis=0) * (1.0 / N)
    var = msq - mu * mu
    scale = gam_ref[...] * jax.lax.rsqrt(var + 1e-5)
    hnew_ref[...] = h_ref[...] + jax.nn.relu(
        (out_ref[...] - mu) * scale + bet_ref[...])


def _bn_apply(out, psum, psq, gamma, beta, h):
    nb = pl.BlockSpec((NBLK, H), lambda i: (i, 0))
    return pl.pallas_call(
        _bn_body,
        grid=(NNB,),
        in_specs=[nb,
                  pl.BlockSpec((NNB, 1, H), lambda i: (0, 0, 0)),
                  pl.BlockSpec((NNB, 1, H), lambda i: (0, 0, 0)),
                  pl.BlockSpec((1, H), lambda i: (0, 0)),
                  pl.BlockSpec((1, H), lambda i: (0, 0)),
                  nb],
        out_specs=nb,
        out_shape=jax.ShapeDtypeStruct((N, H), jnp.float32),
    )(out, psum, psq, gamma.reshape(1, H), beta.reshape(1, H), h)


def _head_body(h_ref, batch_ref, w1_ref, b1_ref, w2_ref, b2_ref, w3_ref, b3_ref,
               out_ref):
    gids = jax.lax.broadcasted_iota(jnp.int32, (NG, N), 0)
    onehot = (batch_ref[...] == gids).astype(jnp.float32)
    gsum = jnp.dot(onehot, h_ref[...], preferred_element_type=jnp.float32)
    gcnt = jnp.sum(onehot, axis=1, keepdims=True)
    g = gsum / jnp.maximum(gcnt, 1.0)
    z = jax.nn.relu(jnp.dot(g, w1_ref[...], preferred_element_type=jnp.float32)
                    + b1_ref[...])
    z = jax.nn.relu(jnp.dot(z, w2_ref[...], preferred_element_type=jnp.float32)
                    + b2_ref[...])
    out_ref[...] = jnp.dot(z, w3_ref[...], preferred_element_type=jnp.float32) \
        + b3_ref[...]


def _head(h, batch, fc1_W, fc1_b, fc2_W, fc2_b, fc3_W, fc3_b):
    full = lambda s: pl.BlockSpec(s, lambda: (0,) * len(s))
    return pl.pallas_call(
        _head_body,
        in_specs=[full((N, H)), full((1, N)),
                  full(fc1_W.shape), full((1, fc1_b.shape[0])),
                  full(fc2_W.shape), full((1, fc2_b.shape[0])),
                  full(fc3_W.shape), full((1, fc3_b.shape[0]))],
        out_specs=full((NG, 10)),
        out_shape=jax.ShapeDtypeStruct((NG, 10), jnp.float32),
    )(h, batch.reshape(1, N), fc1_W, fc1_b.reshape(1, -1),
      fc2_W, fc2_b.reshape(1, -1), fc3_W, fc3_b.reshape(1, -1))


def kernel(x, edge_index, batch, edge_attr, W_ne, b_ne, W_ee, b_ee, W_conv,
           b_conv, bn_gamma, bn_beta, fc1_W, fc1_b, fc2_W, fc2_b, fc3_W, fc3_b):
    src, dst = edge_index[0], edge_index[1]
    h = _matmul_bias(x, W_ne, b_ne, NBLK)
    ea = _matmul_bias(edge_attr, W_ee, b_ee, EBLK)

    deg = jax.ops.segment_sum(jnp.ones((E,), jnp.float32), dst, num_segments=N)
    deg2 = deg.reshape(N, 1)
    cntc = jnp.maximum(deg2, 1.0)
    dmask = deg2 > 0

    ea_sum = jax.ops.segment_sum(ea, dst, num_segments=N)
    ea_min = jax.ops.segment_min(ea, dst, num_segments=N)
    ea_max = jax.ops.segment_max(ea, dst, num_segments=N)
    ea_ssq = jax.ops.segment_sum(ea * ea, dst, num_segments=N)
    ea_mean = ea_sum / cntc
    ea_std = jnp.sqrt(jax.nn.relu(ea_ssq / cntc - ea_mean * ea_mean) + 1e-5)
    e4 = jnp.concatenate([ea_mean, jnp.where(dmask, ea_min, 0.0),
                          jnp.where(dmask, ea_max, 0.0), ea_std], axis=1)

    wq = W_conv.reshape(NLAYERS, 3, 4, 3, H, H)
    wdst = jnp.transpose(wq[:, :, :3, 0].sum(2), (0, 2, 1, 3)).reshape(NLAYERS, H, 3 * H)
    wsrc = jnp.transpose(wq[:, :, :, 1], (0, 2, 3, 1, 4)).reshape(NLAYERS, 4 * H, 3 * H)
    wea = jnp.transpose(wq[:, :, :, 2], (0, 2, 3, 1, 4)).reshape(NLAYERS, 4 * H, 3 * H)
    cs = jnp.float32(np.sqrt(1e-5)) * wq[:, :, 3, 0].sum(axis=2)
    cs = cs.at[:, 0].add(b_conv)

    for i in range(NLAYERS):
        g = h[src]
        ssum = jax.ops.segment_sum(g, dst, num_segments=N)
        smin = jax.ops.segment_min(g, dst, num_segments=N)
        smax = jax.ops.segment_max(g, dst, num_segments=N)
        sssq = jax.ops.segment_sum(g * g, dst, num_segments=N)
        out, psum, psq = _layer_matmul(h, deg2, ssum, smin, smax, sssq, e4,
                                       wdst[i], wsrc[i], wea[i], cs[i])
        h = _bn_apply(out, psum, psq, bn_gamma[i], bn_beta[i], h)

    return _head(h, batch, fc1_W, fc1_b, fc2_W, fc2_b, fc3_W, fc3_b)
